# baseline (speedup 1.0000x reference)
import functools

import jax
import jax.numpy as jnp
from jax import lax
from jax.experimental import pallas as pl
from jax.experimental.pallas import tpu as pltpu
import jax.experimental.pallas.tpu_sc as plsc

B_CELLS = 64
NSLOT = 2


def _stage_a_body(x_ref, wa_ref, wc_ref, ca_ref, a_ref, c_ref):
    xb = x_ref[...]
    nrm = jnp.sqrt(jnp.sum(xb * xb, axis=1, keepdims=True)) + 1e-12
    h = xb / nrm
    a_ref[...] = (
        jnp.dot(h, wa_ref[...], preferred_element_type=jnp.float32) + ca_ref[...]
    ).astype(jnp.bfloat16)
    c_ref[...] = jnp.dot(
        h, wc_ref[...], preferred_element_type=jnp.float32
    ).astype(jnp.bfloat16)


def _stage_a(x, wa, wc, ca, blk):
    n, d = x.shape
    grid = n // blk
    return pl.pallas_call(
        _stage_a_body,
        grid=(grid,),
        in_specs=[
            pl.BlockSpec((blk, d), lambda i: (i, 0)),
            pl.BlockSpec((d, d), lambda i: (0, 0)),
            pl.BlockSpec((d, d), lambda i: (0, 0)),
            pl.BlockSpec((1, d), lambda i: (0, 0)),
        ],
        out_specs=[
            pl.BlockSpec((blk, d), lambda i: (i, 0)),
            pl.BlockSpec((blk, d), lambda i: (i, 0)),
        ],
        out_shape=[
            jax.ShapeDtypeStruct((n, d), jnp.bfloat16),
            jax.ShapeDtypeStruct((n, d), jnp.bfloat16),
        ],
    )(x, wa, wc, ca)


def _stage_b(a2v, c2v, batch, dst, src, *, epw):
    n, dw = a2v.shape
    e = dst.shape[0]
    ch = 128
    npairs = epw // (2 * ch)
    tail = epw - npairs * 2 * ch
    mesh = plsc.VectorSubcoreMesh(core_axis_name="c", subcore_axis_name="s", num_cores=2, num_subcores=16)

    def relu_add(ab, cb_, rb, nedge):
        @pl.loop(0, nedge)
        def _edge(ee):
            for cc in range(dw // 16):
                av = plsc.bitcast(ab[ee, pl.ds(cc * 16, 16)], jnp.bfloat16)
                cv = plsc.bitcast(cb_[ee, pl.ds(cc * 16, 16)], jnp.bfloat16)
                rv = jnp.maximum(av + cv, jnp.bfloat16(0.0))
                rb[ee, pl.ds(cc * 16, 16)] = plsc.bitcast(rv, jnp.int32)

    @functools.partial(
        pl.kernel,
        out_type=[
            jax.ShapeDtypeStruct((e, dw), jnp.int32),
            jax.ShapeDtypeStruct((e,), jnp.int32),
        ],
        mesh=mesh,
        compiler_params=pltpu.CompilerParams(
            needs_layout_passes=False, use_tc_tiling_on_sc=False),
        scratch_types=[
            pltpu.VMEM((epw,), jnp.int32),
            pltpu.VMEM((epw,), jnp.int32),
            pltpu.VMEM((ch, dw), jnp.int32),
            pltpu.VMEM((ch, dw), jnp.int32),
            pltpu.VMEM((ch, dw), jnp.int32),
            pltpu.VMEM((ch, dw), jnp.int32),
            pltpu.VMEM((ch, dw), jnp.int32),
            pltpu.VMEM((ch, dw), jnp.int32),
            pltpu.VMEM((epw,), jnp.int32),
            pltpu.SemaphoreType.DMA,
            pltpu.SemaphoreType.DMA,
            pltpu.SemaphoreType.DMA,
            pltpu.SemaphoreType.DMA,
            pltpu.SemaphoreType.DMA,
            pltpu.SemaphoreType.DMA,
            pltpu.SemaphoreType.DMA,
        ],
    )
    def k(a2_h, c2_h, batch_h, dst_h, src_h, r_h, cb_h,
          idx_d, idx_s, abuf0, abuf1, cbuf0, cbuf1, rbuf0, rbuf1, cbv,
          sa0, sc0, sa1, sc1, so0, so1, scb):
        wid = lax.axis_index("s") * 2 + lax.axis_index("c")
        ebase = wid * epw
        pltpu.sync_copy(dst_h.at[pl.ds(ebase, epw)], idx_d)
        pltpu.sync_copy(src_h.at[pl.ds(ebase, epw)], idx_s)

        abufs = (abuf0, abuf1)
        cbufs = (cbuf0, cbuf1)
        rbufs = (rbuf0, rbuf1)
        sas = (sa0, sa1)
        scs = (sc0, sc1)
        sos = (so0, so1)

        def issue_gathers(k_, par):
            i0 = k_ * ch
            pltpu.async_copy(a2_h.at[idx_d.at[pl.ds(i0, ch)]],
                             abufs[par], sas[par])
            pltpu.async_copy(c2_h.at[idx_s.at[pl.ds(i0, ch)]],
                             cbufs[par], scs[par])
            pltpu.async_copy(batch_h.at[idx_d.at[pl.ds(i0, ch)]],
                             cbv.at[pl.ds(i0, ch)], scb)

        def wait_gathers(par):
            pltpu.make_async_copy(a2_h.at[pl.ds(0, ch)], abufs[par],
                                  sas[par]).wait()
            pltpu.make_async_copy(c2_h.at[pl.ds(0, ch)], cbufs[par],
                                  scs[par]).wait()

        def issue_out(k_, par):
            i0 = k_ * ch
            pltpu.async_copy(rbufs[par],
                             r_h.at[pl.ds(ebase + i0, ch)], sos[par])

        def wait_out(par):
            pltpu.make_async_copy(rbufs[par], r_h.at[pl.ds(0, ch)],
                                  sos[par]).wait()

        issue_gathers(0, 0)
        issue_gathers(1, 1)

        @pl.loop(0, npairs)
        def _pair(p):
            a = p * 2

            wait_gathers(0)

            @pl.when(p > 0)
            def _():
                wait_out(0)
            relu_add(abuf0, cbuf0, rbuf0, ch)
            issue_out(a, 0)

            @pl.when(p + 1 < npairs)
            def _():
                issue_gathers(a + 2, 0)

            wait_gathers(1)

            @pl.when(p > 0)
            def _():
                wait_out(1)
            relu_add(abuf1, cbuf1, rbuf1, ch)
            issue_out(a + 1, 1)

            @pl.when(p + 1 < npairs)
            def _():
                issue_gathers(a + 3, 1)

        wait_out(0)
        wait_out(1)

        if tail:
            t0 = npairs * 2 * ch
            pltpu.async_copy(
                a2_h.at[idx_d.at[pl.ds(t0, tail)]],
                abuf0.at[pl.ds(0, tail)], sas[0])
            pltpu.async_copy(
                c2_h.at[idx_s.at[pl.ds(t0, tail)]],
                cbuf0.at[pl.ds(0, tail)], scs[0])
            pltpu.async_copy(batch_h.at[idx_d.at[pl.ds(t0, tail)]],
                             cbv.at[pl.ds(t0, tail)], scb)
            pltpu.make_async_copy(a2_h.at[pl.ds(0, tail)],
                                  abuf0.at[pl.ds(0, tail)], sas[0]).wait()
            pltpu.make_async_copy(c2_h.at[pl.ds(0, tail)],
                                  cbuf0.at[pl.ds(0, tail)], scs[0]).wait()
            relu_add(abuf0, cbuf0, rbuf0, tail)
            pltpu.sync_copy(rbuf0.at[pl.ds(0, tail)],
                            r_h.at[pl.ds(ebase + t0, tail)])

        pltpu.make_async_copy(batch_h.at[pl.ds(0, epw)], cbv, scb).wait()
        pltpu.sync_copy(cbv, cb_h.at[pl.ds(ebase, epw)])

    return k(a2v, c2v, batch, dst, src)


def _stage_c_body(r_ref, w_ref, b_ref, s_ref):
    acc = jnp.dot(r_ref[...], w_ref[...], preferred_element_type=jnp.float32)
    s_ref[...] = jnp.maximum(acc + b_ref[...], 0.0).astype(jnp.bfloat16)


def _stage_c(r, w2p, b2p, blk):
    e, d = r.shape
    grid = e // blk
    return pl.pallas_call(
        _stage_c_body,
        grid=(grid,),
        in_specs=[
            pl.BlockSpec((blk, d), lambda i: (i, 0)),
            pl.BlockSpec((d, d), lambda i: (0, 0)),
            pl.BlockSpec((1, d), lambda i: (0, 0)),
        ],
        out_specs=pl.BlockSpec((blk, d), lambda i: (i, 0)),
        out_shape=jax.ShapeDtypeStruct((e, d), jnp.bfloat16),
    )(r, w2p, b2p)


def _stage_d(sv_words, cb, *, epw, ch):
    e, dw = sv_words.shape
    assert ch % 16 == 0 and epw % ch == 0, "chunk must cover whole 16-groups"
    nch = epw // ch
    npairs = nch // 2
    nw = 32
    mesh = plsc.VectorSubcoreMesh(core_axis_name="c", subcore_axis_name="s", num_cores=2, num_subcores=16)

    @functools.partial(
        pl.kernel,
        out_type=jax.ShapeDtypeStruct((nw * NSLOT, B_CELLS, dw), jnp.int32),
        mesh=mesh,
        compiler_params=pltpu.CompilerParams(
            needs_layout_passes=False, use_tc_tiling_on_sc=False),
        scratch_types=[
            pltpu.VMEM((ch, dw), jnp.int32),
            pltpu.VMEM((ch, dw), jnp.int32),
            pltpu.VMEM((ch,), jnp.int32),
            pltpu.VMEM((ch,), jnp.int32),
            pltpu.VMEM((NSLOT * B_CELLS, dw), jnp.int32),
            pltpu.SemaphoreType.DMA,
            pltpu.SemaphoreType.DMA,
            pltpu.SemaphoreType.DMA,
            pltpu.SemaphoreType.DMA,
        ],
    )
    def k(s_h, cb_h, out_h, sbuf0, sbuf1, cbv0, cbv1, acc2,
          ss0, ss1, sb0, sb1):
        wid = lax.axis_index("s") * 2 + lax.axis_index("c")
        ebase = wid * epw
        sbufs = (sbuf0, sbuf1)
        cbvs = (cbv0, cbv1)
        sss = (ss0, ss1)
        sbs = (sb0, sb1)

        @pl.loop(0, NSLOT * B_CELLS)
        def _zrow(rr):
            for cc in range(dw // 16):
                acc2[rr, pl.ds(cc * 16, 16)] = jnp.zeros((16,), jnp.int32)

        def issue(k_, par):
            i0 = ebase + k_ * ch
            pltpu.async_copy(s_h.at[pl.ds(i0, ch)], sbufs[par], sss[par])
            pltpu.async_copy(cb_h.at[pl.ds(i0, ch)], cbvs[par], sbs[par])

        def wait(par):
            pltpu.make_async_copy(s_h.at[pl.ds(0, ch)], sbufs[par],
                                  sss[par]).wait()
            pltpu.make_async_copy(cb_h.at[pl.ds(0, ch)], cbvs[par],
                                  sbs[par]).wait()

        def consume(par):
            @pl.loop(0, ch // 16)
            def _grp(gg):
                cb16 = cbvs[par][pl.ds(gg * 16, 16)]
                for j in range(16):
                    cj = cb16[j] + (j % NSLOT) * B_CELLS
                    ee = gg * 16 + j
                    for cc in range(dw // 16):
                        sv = plsc.bitcast(
                            sbufs[par][ee, pl.ds(cc * 16, 16)], jnp.bfloat16)
                        av = plsc.bitcast(
                            acc2[cj, pl.ds(cc * 16, 16)], jnp.bfloat16)
                        acc2[cj, pl.ds(cc * 16, 16)] = plsc.bitcast(
                            jnp.maximum(av, sv), jnp.int32)

        issue(0, 0)
        if nch > 1:
            issue(1, 1)

        @pl.loop(0, npairs)
        def _pair(p):
            a = p * 2
            wait(0)
            consume(0)

            @pl.when(a + 2 < nch)
            def _():
                issue(a + 2, 0)

            wait(1)
            consume(1)

            @pl.when(a + 3 < nch)
            def _():
                issue(a + 3, 1)

        if nch % 2:
            wait(0)
            consume(0)

        for sl in range(NSLOT):
            pltpu.sync_copy(
                acc2.at[pl.ds(sl * B_CELLS, B_CELLS)],
                out_h.at[NSLOT * wid + sl])

    return k(sv_words, cb)


def _stage_e_body(p_ref, wl1_ref, bl1_ref, wl2_ref, bl2_ref, y_ref):
    pooled = jnp.max(p_ref[...].astype(jnp.float32), axis=0)
    y1 = jnp.maximum(
        jnp.dot(pooled, wl1_ref[...], preferred_element_type=jnp.float32)
        + bl1_ref[...],
        0.0,
    )
    y2 = (
        jnp.dot(y1, wl2_ref[...], preferred_element_type=jnp.float32) + bl2_ref[...]
    )
    nrm = jnp.sqrt(jnp.sum(y2 * y2, axis=1, keepdims=True)) + 1e-12
    y_ref[...] = y2 / nrm


def _stage_e(partials, wl1t, bl1, wl2t, bl2):
    nw, b, d = partials.shape
    return pl.pallas_call(
        _stage_e_body,
        in_specs=[
            pl.BlockSpec((nw, b, d), lambda: (0, 0, 0)),
            pl.BlockSpec((d, d), lambda: (0, 0)),
            pl.BlockSpec((1, d), lambda: (0, 0)),
            pl.BlockSpec((d, d), lambda: (0, 0)),
            pl.BlockSpec((1, d), lambda: (0, 0)),
        ],
        out_specs=pl.BlockSpec((b, d), lambda: (0, 0)),
        out_shape=jax.ShapeDtypeStruct((b, d), jnp.float32),
    )(partials, wl1t, bl1, wl2t, bl2)


def kernel(x, edge_index, batch, W1, b1, g1, be1, W2, b2, g2, be2,
           Wl1, bl1, Wl2, bl2):
    n, d = x.shape
    e = edge_index.shape[1]
    nw = 32
    epw = e // nw
    assert e % nw == 0 and n % 8 == 0

    w1a = W1[:, :d]
    w1b = W1[:, d:]
    wa = (w1a - w1b).T * g1[None, :]
    wc = w1b.T * g1[None, :]
    ca = (g1 * b1 + be1)[None, :]
    w2p = ((W2 * g2[:, None]).T).astype(jnp.bfloat16)
    b2p = (g2 * b2 + be2)[None, :]

    a2, c2 = _stage_a(x, wa, wc, ca, blk=400)
    a2v = lax.bitcast_convert_type(a2.reshape(n, d // 2, 2), jnp.int32)
    c2v = lax.bitcast_convert_type(c2.reshape(n, d // 2, 2), jnp.int32)
    dst = edge_index[1]
    src = edge_index[0]
    rv, cb = _stage_b(a2v, c2v, batch, dst, src, epw=epw)
    r = lax.bitcast_convert_type(rv[..., None], jnp.bfloat16).reshape(e, d)
    s = _stage_c(r, w2p, b2p, blk=2000)
    sv_words = lax.bitcast_convert_type(s.reshape(e, d // 2, 2), jnp.int32)
    pw = _stage_d(sv_words, cb, epw=epw, ch=400)
    partials = lax.bitcast_convert_type(pw[..., None], jnp.bfloat16).reshape(
        pw.shape[0], B_CELLS, d)
    y = _stage_e(partials, Wl1.T, bl1[None, :], Wl2.T, bl2[None, :])
    return y


# speedup vs baseline: 2.3619x; 2.3619x over previous
import functools

import jax
import jax.numpy as jnp
from jax import lax
from jax.experimental import pallas as pl
from jax.experimental.pallas import tpu as pltpu
import jax.experimental.pallas.tpu_sc as plsc

B_CELLS = 64


def _stage_a_body(x_ref, wa_ref, wc_ref, ca_ref, a_ref, c_ref):
    xb = x_ref[...]
    nrm = jnp.sqrt(jnp.sum(xb * xb, axis=1, keepdims=True)) + 1e-12
    h = xb / nrm
    a_ref[...] = (
        jnp.dot(h, wa_ref[...], preferred_element_type=jnp.float32) + ca_ref[...]
    )
    c_ref[...] = jnp.dot(h, wc_ref[...], preferred_element_type=jnp.float32)


def _stage_a(x, wa, wc, ca, blk):
    n, d = x.shape
    grid = n // blk
    return pl.pallas_call(
        _stage_a_body,
        grid=(grid,),
        in_specs=[
            pl.BlockSpec((blk, d), lambda i: (i, 0)),
            pl.BlockSpec((d, d), lambda i: (0, 0)),
            pl.BlockSpec((d, d), lambda i: (0, 0)),
            pl.BlockSpec((1, d), lambda i: (0, 0)),
        ],
        out_specs=[
            pl.BlockSpec((blk, d), lambda i: (i, 0)),
            pl.BlockSpec((blk, d), lambda i: (i, 0)),
        ],
        out_shape=[
            jax.ShapeDtypeStruct((n, d), jnp.float32),
            jax.ShapeDtypeStruct((n, d), jnp.float32),
        ],
    )(x, wa, wc, ca)


def _stage_b(a2, c2, batch, dst, src, *, epw, ch):
    n, d = a2.shape
    e = dst.shape[0]
    nch = epw // ch
    mesh = plsc.VectorSubcoreMesh(core_axis_name="c", subcore_axis_name="s")

    @functools.partial(
        pl.kernel,
        out_type=[
            jax.ShapeDtypeStruct((e, d), jnp.float32),
            jax.ShapeDtypeStruct((e,), jnp.int32),
        ],
        mesh=mesh,
        scratch_types=[
            pltpu.VMEM((epw,), jnp.int32),
            pltpu.VMEM((epw,), jnp.int32),
            pltpu.VMEM((ch, d), jnp.float32),
            pltpu.VMEM((ch, d), jnp.float32),
            pltpu.VMEM((ch,), jnp.int32),
            pltpu.SemaphoreType.DMA,
            pltpu.SemaphoreType.DMA,
            pltpu.SemaphoreType.DMA,
        ],
    )
    def k(a2_h, c2_h, batch_h, dst_h, src_h, r_h, cb_h,
          idx_d, idx_s, abuf, cbuf, cbv, sem0, sem1, sem2):
        wid = lax.axis_index("s") * 2 + lax.axis_index("c")
        ebase = wid * epw
        pltpu.sync_copy(dst_h.at[pl.ds(ebase, epw)], idx_d)
        pltpu.sync_copy(src_h.at[pl.ds(ebase, epw)], idx_s)

        @pl.loop(0, nch)
        def _chunk(kk):
            i0 = kk * ch
            di = idx_d.at[pl.ds(i0, ch)]
            si = idx_s.at[pl.ds(i0, ch)]
            d1 = pltpu.async_copy(a2_h.at[di], abuf, sem0)
            d2 = pltpu.async_copy(c2_h.at[si], cbuf, sem1)
            d3 = pltpu.async_copy(batch_h.at[di], cbv, sem2)
            d1.wait()
            d2.wait()

            @pl.loop(0, ch)
            def _edge(ee):
                for cc in range(d // 16):
                    av = abuf[ee, pl.ds(cc * 16, 16)]
                    cv = cbuf[ee, pl.ds(cc * 16, 16)]
                    abuf[ee, pl.ds(cc * 16, 16)] = jnp.maximum(av + cv, 0.0)

            d3.wait()
            pltpu.sync_copy(abuf, r_h.at[pl.ds(ebase + i0, ch)])
            pltpu.sync_copy(cbv, cb_h.at[pl.ds(ebase + i0, ch)])

    return k(a2, c2, batch, dst, src)


def _stage_c_body(r_ref, w_ref, b_ref, s_ref):
    s_ref[...] = jnp.maximum(
        jnp.dot(r_ref[...], w_ref[...], preferred_element_type=jnp.float32)
        + b_ref[...],
        0.0,
    )


def _stage_c(r, w2p, b2p, blk):
    e, d = r.shape
    grid = e // blk
    return pl.pallas_call(
        _stage_c_body,
        grid=(grid,),
        in_specs=[
            pl.BlockSpec((blk, d), lambda i: (i, 0)),
            pl.BlockSpec((d, d), lambda i: (0, 0)),
            pl.BlockSpec((1, d), lambda i: (0, 0)),
        ],
        out_specs=pl.BlockSpec((blk, d), lambda i: (i, 0)),
        out_shape=jax.ShapeDtypeStruct((e, d), jnp.float32),
    )(r, w2p, b2p)


def _stage_d(s, cb, *, epw, ch):
    e, d = s.shape
    nch = epw // ch
    nw = 32
    mesh = plsc.VectorSubcoreMesh(core_axis_name="c", subcore_axis_name="s")

    @functools.partial(
        pl.kernel,
        out_type=jax.ShapeDtypeStruct((nw, B_CELLS, d), jnp.float32),
        mesh=mesh,
        compiler_params=pltpu.CompilerParams(needs_layout_passes=False),
        scratch_types=[
            pltpu.VMEM((ch, d), jnp.float32),
            pltpu.VMEM((ch,), jnp.int32),
            pltpu.VMEM((B_CELLS, d), jnp.float32),
        ],
    )
    def k(s_h, cb_h, out_h, sbuf, cbv, acc):
        lane = lax.iota(jnp.int32, 16)
        wid = lax.axis_index("s") * 2 + lax.axis_index("c")
        ebase = wid * epw

        @pl.loop(0, B_CELLS)
        def _zrow(rr):
            for cc in range(d // 16):
                acc[rr, pl.ds(cc * 16, 16)] = jnp.zeros((16,), jnp.float32)

        @pl.loop(0, nch)
        def _chunk(kk):
            i0 = ebase + kk * ch
            pltpu.sync_copy(s_h.at[pl.ds(i0, ch)], sbuf)
            pltpu.sync_copy(cb_h.at[pl.ds(i0, ch)], cbv)

            @pl.loop(0, ch // 16)
            def _grp(gg):
                cb16 = cbv[pl.ds(gg * 16, 16)]
                for j in range(16):
                    cj = jnp.sum(jnp.where(lane == j, cb16, 0))
                    ee = gg * 16 + j
                    for cc in range(d // 16):
                        sv = sbuf[ee, pl.ds(cc * 16, 16)]
                        av = acc[cj, pl.ds(cc * 16, 16)]
                        acc[cj, pl.ds(cc * 16, 16)] = jnp.maximum(av, sv)

        pltpu.sync_copy(acc, out_h.at[wid])

    return k(s, cb)


def _stage_e_body(p_ref, wl1_ref, bl1_ref, wl2_ref, bl2_ref, y_ref):
    pooled = jnp.max(p_ref[...], axis=0)
    y1 = jnp.maximum(
        jnp.dot(pooled, wl1_ref[...], preferred_element_type=jnp.float32)
        + bl1_ref[...],
        0.0,
    )
    y2 = (
        jnp.dot(y1, wl2_ref[...], preferred_element_type=jnp.float32) + bl2_ref[...]
    )
    nrm = jnp.sqrt(jnp.sum(y2 * y2, axis=1, keepdims=True)) + 1e-12
    y_ref[...] = y2 / nrm


def _stage_e(partials, wl1t, bl1, wl2t, bl2):
    nw, b, d = partials.shape
    return pl.pallas_call(
        _stage_e_body,
        in_specs=[
            pl.BlockSpec((nw, b, d), lambda: (0, 0, 0)),
            pl.BlockSpec((d, d), lambda: (0, 0)),
            pl.BlockSpec((1, d), lambda: (0, 0)),
            pl.BlockSpec((d, d), lambda: (0, 0)),
            pl.BlockSpec((1, d), lambda: (0, 0)),
        ],
        out_specs=pl.BlockSpec((b, d), lambda: (0, 0)),
        out_shape=jax.ShapeDtypeStruct((b, d), jnp.float32),
    )(partials, wl1t, bl1, wl2t, bl2)


def kernel(x, edge_index, batch, W1, b1, g1, be1, W2, b2, g2, be2,
           Wl1, bl1, Wl2, bl2):
    n, d = x.shape
    e = edge_index.shape[1]
    nw = 32
    epw = e // nw
    assert e % nw == 0 and n % 8 == 0

    w1a = W1[:, :d]
    w1b = W1[:, d:]
    wa = (w1a - w1b).T * g1[None, :]
    wc = w1b.T * g1[None, :]
    ca = (g1 * b1 + be1)[None, :]
    w2p = (W2 * g2[:, None]).T
    b2p = (g2 * b2 + be2)[None, :]

    a2, c2 = _stage_a(x, wa, wc, ca, blk=400)
    dst = edge_index[1]
    src = edge_index[0]
    r, cb = _stage_b(a2, c2, batch, dst, src, epw=epw, ch=80)
    s = _stage_c(r, w2p, b2p, blk=2000)
    partials = _stage_d(s, cb, epw=epw, ch=400)
    y = _stage_e(partials, Wl1.T, bl1[None, :], Wl2.T, bl2[None, :])
    return y
